# Initial kernel scaffold; baseline (speedup 1.0000x reference)
import jax
import jax.numpy as jnp
from jax.experimental import pallas as pl


def kernel(x, hour_table, weekday_table, day_table, month_table):
    raise NotImplementedError("write your pallas kernel here")


# speedup vs baseline: 7.6911x; 7.6911x over previous
import functools

import jax
import jax.numpy as jnp
from jax import lax
from jax.experimental import pallas as pl
from jax.experimental.pallas import tpu as pltpu
from jax.experimental.pallas import tpu_sc as plsc

D = 128
N_ROWS = 4096 * 50
NC, NS = 2, 16
NW = NC * NS
ROWS_PER_W = N_ROWS // NW
BLK = 256
N_BLK = ROWS_PER_W // BLK


def _div_const(n, c):
    q = (n.astype(jnp.float32) * jnp.float32(1.0 / c)).astype(jnp.int32)
    r = n - q * c
    q = jnp.where(r < 0, q - 1, q)
    r = jnp.where(r < 0, r + c, r)
    q = jnp.where(r >= c, q + 1, q)
    return q


def _mod_const(n, c):
    return n - _div_const(n, c) * c


def _combine_tables_kernel(hour_ref, weekday_ref, day_ref, month_ref,
                           t1_ref, t2_ref):
    m = month_ref[...]
    d = day_ref[...]
    for h in range(24):
        t1_ref[h * 12:(h + 1) * 12, :] = m[:12, :] + hour_ref[h:h + 1, :]
    for w in range(7):
        t2_ref[w * 31:(w + 1) * 31, :] = d[:31, :] + weekday_ref[w:w + 1, :]


def _combine_tables(hour_table, weekday_table, day_table, month_table):
    return pl.pallas_call(
        _combine_tables_kernel,
        out_shape=(
            jax.ShapeDtypeStruct((288, D), jnp.float32),
            jax.ShapeDtypeStruct((217, D), jnp.float32),
        ),
    )(hour_table, weekday_table, day_table, month_table)


def _sc_kernel(a_hbm, b_hbm, t1_hbm, t2_hbm, out_hbm,
               a_v, b_v, i1_v, i2_v, g1_v, g2_v, sem):
    i32 = jnp.int32
    wid = lax.axis_index("s") * i32(NC) + lax.axis_index("c")

    @pl.loop(jnp.int32(0), jnp.int32(N_BLK))
    def _blk(j):
        base = wid * i32(ROWS_PER_W) + j.astype(i32) * i32(BLK)
        pltpu.sync_copy(a_hbm.at[pl.ds(base, BLK)], a_v)
        pltpu.sync_copy(b_hbm.at[pl.ds(base, BLK)], b_v)

        @pl.loop(jnp.int32(0), jnp.int32(BLK // 16))
        def _idx(g):
            sl = pl.ds(g.astype(i32) * i32(16), 16)
            a = a_v[sl]
            b = b_v[sl]
            t = a * 216 + b
            y = a * 16777 + _div_const(t, 1000)
            d = _div_const(y, 86400)
            hour = _div_const(_mod_const(y, 1440), 60)
            month = _mod_const(_div_const(y, 2629800), 12)
            i1_v[sl] = hour * 12 + month
            i2_v[sl] = _mod_const(d, 7) * 31 + _mod_const(d, 31)

        cp1 = pltpu.async_copy(t1_hbm.at[i1_v], g1_v, sem)
        cp2 = pltpu.async_copy(t2_hbm.at[i2_v], g2_v, sem)
        cp1.wait()
        cp2.wait()

        @pl.loop(jnp.int32(0), jnp.int32(BLK))
        def _add(r):
            ri = r.astype(i32)
            for c in range(D // 16):
                sl = pl.ds(c * 16, 16)
                g1_v[ri, sl] = g1_v[ri, sl] + g2_v[ri, sl]

        pltpu.sync_copy(g1_v, out_hbm.at[pl.ds(base, BLK)])


def kernel(x, hour_table, weekday_table, day_table, month_table):
    xf = x.reshape(-1)
    a = (xf >> 24).astype(jnp.int32)
    b = (xf & 0xFFFFFF).astype(jnp.int32)

    t1, t2 = _combine_tables(hour_table, weekday_table, day_table, month_table)

    mesh = plsc.VectorSubcoreMesh(core_axis_name="c", subcore_axis_name="s")
    sc = pl.kernel(
        _sc_kernel,
        mesh=mesh,
        out_type=jax.ShapeDtypeStruct((N_ROWS, D), jnp.float32),
        scratch_types=[
            pltpu.VMEM((BLK,), jnp.int32),
            pltpu.VMEM((BLK,), jnp.int32),
            pltpu.VMEM((BLK,), jnp.int32),
            pltpu.VMEM((BLK,), jnp.int32),
            pltpu.VMEM((BLK, D), jnp.float32),
            pltpu.VMEM((BLK, D), jnp.float32),
            pltpu.SemaphoreType.DMA,
        ],
    )
    out = sc(a, b, t1, t2)
    return out.reshape(x.shape[0], x.shape[1], D)


# speedup vs baseline: 10.3214x; 1.3420x over previous
import functools

import jax
import jax.numpy as jnp
from jax import lax
from jax.experimental import pallas as pl
from jax.experimental.pallas import tpu as pltpu
from jax.experimental.pallas import tpu_sc as plsc

D = 128
L_SEQ = 50
N_ROWS = 4096 * L_SEQ
NC, NS = 2, 16
NW = NC * NS
ROWS_PER_W = N_ROWS // NW
BLK = 400
N_BLK = ROWS_PER_W // BLK


def _div_const(n, c):
    q = (n.astype(jnp.float32) * jnp.float32(1.0 / c)).astype(jnp.int32)
    r = n - q * c
    q = jnp.where(r < 0, q - 1, q)
    r = jnp.where(r < 0, r + c, r)
    q = jnp.where(r >= c, q + 1, q)
    return q


def _mod_const(n, c):
    return n - _div_const(n, c) * c


def _combine_tables_kernel(hour_ref, weekday_ref, day_ref, month_ref,
                           t1_ref, t2_ref):
    m = month_ref[...]
    d = day_ref[...]
    for h in range(24):
        t1_ref[h * 12:(h + 1) * 12, :] = m[:12, :] + hour_ref[h:h + 1, :]
    for w in range(7):
        t2_ref[w * 31:(w + 1) * 31, :] = d[:31, :] + weekday_ref[w:w + 1, :]


def _combine_tables(hour_table, weekday_table, day_table, month_table):
    return pl.pallas_call(
        _combine_tables_kernel,
        out_shape=(
            jax.ShapeDtypeStruct((288, D), jnp.float32),
            jax.ShapeDtypeStruct((217, D), jnp.float32),
        ),
    )(hour_table, weekday_table, day_table, month_table)


def _sc_kernel(a_hbm, b_hbm, t1_hbm, t2_hbm, out_hbm,
               a_v, b_v, i1_v, i2_v, g1_v, g2_v, sem):
    i32 = jnp.int32
    wid = lax.axis_index("s") * i32(NC) + lax.axis_index("c")

    @pl.loop(jnp.int32(0), jnp.int32(N_BLK))
    def _blk(j):
        base = wid * i32(ROWS_PER_W) + j.astype(i32) * i32(BLK)
        pltpu.sync_copy(a_hbm.at[pl.ds(base, BLK)], a_v)
        pltpu.sync_copy(b_hbm.at[pl.ds(base, BLK)], b_v)
        base_batch = wid * i32(ROWS_PER_W // L_SEQ) + j.astype(i32) * i32(BLK // L_SEQ)

        @pl.loop(jnp.int32(0), jnp.int32(BLK // 16))
        def _idx(g):
            sl = pl.ds(g.astype(i32) * i32(16), 16)
            a = a_v[sl]
            b = b_v[sl]
            t = a * 216 + b
            y = a * 16777 + _div_const(t, 1000)
            d = _div_const(y, 86400)
            hour = _div_const(_mod_const(y, 1440), 60)
            month = _mod_const(_div_const(y, 2629800), 12)
            i1_v[sl] = hour * 12 + month
            i2_v[sl] = _mod_const(d, 7) * 31 + _mod_const(d, 31)

        cp1 = pltpu.async_copy(t1_hbm.at[i1_v], g1_v, sem)
        cp2 = pltpu.async_copy(t2_hbm.at[i2_v], g2_v, sem)
        cp1.wait()
        cp2.wait()

        @pl.loop(jnp.int32(0), jnp.int32(BLK))
        def _add(r):
            ri = r.astype(i32)
            for c in range(D // 16):
                sl = pl.ds(c * 16, 16)
                g1_v[ri, sl] = g1_v[ri, sl] + g2_v[ri, sl]

        for i in range(BLK // L_SEQ):
            pltpu.sync_copy(g1_v.at[pl.ds(i * L_SEQ, L_SEQ)],
                            out_hbm.at[base_batch + i32(i)])


def kernel(x, hour_table, weekday_table, day_table, month_table):
    xf = x.reshape(-1)
    a = (xf >> 24).astype(jnp.int32)
    b = (xf & 0xFFFFFF).astype(jnp.int32)

    t1, t2 = _combine_tables(hour_table, weekday_table, day_table, month_table)

    mesh = plsc.VectorSubcoreMesh(core_axis_name="c", subcore_axis_name="s")
    sc = pl.kernel(
        _sc_kernel,
        mesh=mesh,
        out_type=jax.ShapeDtypeStruct((x.shape[0], L_SEQ, D), jnp.float32),
        scratch_types=[
            pltpu.VMEM((BLK,), jnp.int32),
            pltpu.VMEM((BLK,), jnp.int32),
            pltpu.VMEM((BLK,), jnp.int32),
            pltpu.VMEM((BLK,), jnp.int32),
            pltpu.VMEM((BLK, D), jnp.float32),
            pltpu.VMEM((BLK, D), jnp.float32),
            pltpu.SemaphoreType.DMA,
        ],
    )
    return sc(a, b, t1, t2)
